# baseline (speedup 1.0000x reference)
import functools

import jax
import jax.numpy as jnp
import numpy as np
from jax.experimental import pallas as pl
from jax.experimental.pallas import tpu as pltpu
from jax.experimental.pallas import tpu_sc as plsc

HEADS = 4
K_DIM = 256
KNN = 32
N_KEYS = 512
IN_DIM = 1024
OUT_DIM = 512
HALF = K_DIM // 2

BLK = 128
NEG = np.float32(-3.0e38)
BIGI = np.int32(2**30)

_NB = [KNN // (a + 1) for a in range(KNN)]
_NCAND = sum(_NB)
_CPAD = 128 - _NCAND


def _argmax_tree(v, i, aux=None):
    n = v.shape[0]
    while n > 1:
        h = n // 2
        take = v[:h] >= v[h:]
        v = jnp.where(take, v[:h], v[h:])
        i = jnp.where(take, i[:h], i[h:])
        if aux is not None:
            aux = jnp.where(take, aux[:h], aux[h:])
        n = h
    return (v, i) if aux is None else (v, i, aux)


def _topk_iter(s, k, iota):
    vals, idxs = [], []
    for _ in range(k):
        m, pos = _argmax_tree(s, iota)
        vals.append(m)
        idxs.append(pos)
        s = jnp.where(iota == pos, NEG, s)
    return jnp.concatenate(vals, axis=0), jnp.concatenate(idxs, axis=0)


def _stage_a_body(x_ref, wq_ref, bq_ref, keys_ref, idx_ref, w_ref):
    x = x_ref[...]
    q = jax.lax.dot_general(
        x, wq_ref[...], (((1,), (1,)), ((), ())),
        preferred_element_type=jnp.float32,
    ) + bq_ref[...]

    iota_n = jax.lax.broadcasted_iota(jnp.int32, (N_KEYS, BLK), 0)
    iota_c = jax.lax.broadcasted_iota(jnp.int32, (128, BLK), 0)

    w_heads, i_heads = [], []
    for h in range(HEADS):
        tv, ti = [], []
        for p in range(2):
            off = h * K_DIM + p * HALF
            qh = q[:, off:off + HALF]
            sT = jax.lax.dot_general(
                keys_ref[h, p], qh, (((1,), (1,)), ((), ())),
                preferred_element_type=jnp.float32,
                    )
            v, i = _topk_iter(sT, KNN, iota_n)
            tv.append(v)
            ti.append(i)
        (v1, v2), (i1, i2) = tv, ti
        cv, ci = [], []
        for a in range(KNN):
            nb = _NB[a]
            cv.append(v1[a:a + 1] + v2[:nb])
            ci.append(i1[a:a + 1] * N_KEYS + i2[:nb])
        cv.append(jnp.full((_CPAD, BLK), NEG, jnp.float32))
        ci.append(jnp.zeros((_CPAD, BLK), jnp.int32))
        cv = jnp.concatenate(cv, axis=0)
        ci = jnp.concatenate(ci, axis=0)

        svals, gidxs = [], []
        for _ in range(KNN):
            m, pos, gi = _argmax_tree(cv, iota_c, ci)
            svals.append(m)
            gidxs.append(gi)
            cv = jnp.where(iota_c == pos, NEG, cv)
        sc = jnp.concatenate(svals, axis=0)
        gi = jnp.concatenate(gidxs, axis=0)
        e = jnp.exp(sc - sc[0:1])
        wgt = e / jnp.sum(e, axis=0, keepdims=True)
        w_heads.append(wgt)
        i_heads.append(gi)

    w_all = jnp.concatenate(w_heads, axis=0)
    i_all = jnp.concatenate(i_heads, axis=0)
    idx_ref[...] = i_all.T
    w_ref[...] = w_all.T


def _stage_a(x2d, wq, bq2d, keys):
    bs = x2d.shape[0]
    grid = bs // BLK
    return pl.pallas_call(
        _stage_a_body,
        grid=(grid,),
        in_specs=[
            pl.BlockSpec((BLK, IN_DIM), lambda i: (i, 0)),
            pl.BlockSpec((HEADS * K_DIM, IN_DIM), lambda i: (0, 0)),
            pl.BlockSpec((1, HEADS * K_DIM), lambda i: (0, 0)),
            pl.BlockSpec((HEADS, 2, N_KEYS, HALF), lambda i: (0, 0, 0, 0)),
        ],
        out_specs=[
            pl.BlockSpec((BLK, HEADS * KNN), lambda i: (i, 0)),
            pl.BlockSpec((BLK, HEADS * KNN), lambda i: (i, 0)),
        ],
        out_shape=[
            jax.ShapeDtypeStruct((bs, HEADS * KNN), jnp.int32),
            jax.ShapeDtypeStruct((bs, HEADS * KNN), jnp.float32),
        ],
    )(x2d, wq, bq2d, keys)


NW = 32
F = HEADS * KNN


G = 32
HF = F // 2


def _stage_b(values, idx, w):
    bs = idx.shape[0]
    rpw = bs // NW
    ngrp = rpw // G
    mesh = plsc.VectorSubcoreMesh(core_axis_name="c", subcore_axis_name="s")

    @functools.partial(
        pl.kernel,
        out_type=jax.ShapeDtypeStruct((bs, OUT_DIM), jnp.float32),
        mesh=mesh,
        scratch_types=[
            pltpu.VMEM((G, F), jnp.int32),
            pltpu.VMEM((G, F), jnp.float32),
            pltpu.VMEM((G, OUT_DIM), jnp.float32),
            pltpu.VMEM((HF, OUT_DIM), jnp.float32),
            pltpu.VMEM((HF, OUT_DIM), jnp.float32),
            pltpu.SemaphoreType.DMA,
            pltpu.SemaphoreType.DMA,
        ],
        compiler_params=pltpu.CompilerParams(needs_layout_passes=False),
    )
    def bag(values_hbm, idx_hbm, w_hbm, out_hbm, idx_blk, w_blk, out_blk,
            buf0, buf1, sem0, sem1):
        wid = jax.lax.axis_index("s") * 2 + jax.lax.axis_index("c")
        base = wid * rpw

        def gather(r, h, buf, sem):
            return pltpu.make_async_copy(
                values_hbm.at[idx_blk.at[r, pl.ds(h * HF, HF)]], buf, sem)

        def mac_half(r, joff, buf, acc):
            def mac(j, acc):
                wj = plsc.load_gather(
                    w_blk, [jnp.full((16,), r, jnp.int32),
                            jnp.full((16,), joff + j, jnp.int32)])
                return tuple(
                    acc[c] + wj * buf[j, pl.ds(c * 16, 16)]
                    for c in range(OUT_DIM // 16)
                )
            return jax.lax.fori_loop(0, HF, mac, acc)

        def group_body(g, _):
            gbase = base + g * G
            pltpu.sync_copy(idx_hbm.at[pl.ds(gbase, G)], idx_blk)
            pltpu.sync_copy(w_hbm.at[pl.ds(gbase, G)], w_blk)
            gather(0, 0, buf0, sem0).start()
            gather(0, 1, buf1, sem1).start()

            def row_body(r, _):
                zero = tuple(jnp.zeros((16,), jnp.float32)
                             for _ in range(OUT_DIM // 16))
                gather(r, 0, buf0, sem0).wait()
                acc = mac_half(r, 0, buf0, zero)

                @pl.when(r + 1 < G)
                def _():
                    gather(r + 1, 0, buf0, sem0).start()

                gather(r, 1, buf1, sem1).wait()
                acc = mac_half(r, HF, buf1, acc)

                @pl.when(r + 1 < G)
                def _():
                    gather(r + 1, 1, buf1, sem1).start()

                for c in range(OUT_DIM // 16):
                    out_blk[r, pl.ds(c * 16, 16)] = acc[c]
                return ()

            jax.lax.fori_loop(0, G, row_body, ())
            pltpu.sync_copy(out_blk, out_hbm.at[pl.ds(gbase, G)])
            return ()

        jax.lax.fori_loop(0, ngrp, group_body, ())

    return bag(values, idx, w)


NCHUNK = 16


def kernel(x, Wq, bq, keys, values):
    prefix = x.shape[:-1]
    x2d = x.reshape(-1, IN_DIM)
    bs = x2d.shape[0]
    cs = bs // NCHUNK
    bq2d = bq.reshape(1, -1)
    outs = []
    for c in range(NCHUNK):
        xc = jax.lax.slice_in_dim(x2d, c * cs, (c + 1) * cs, axis=0)
        idx, w = _stage_a(xc, Wq, bq2d, keys)
        outs.append(_stage_b(values, idx, w))
    out = jnp.concatenate(outs, axis=0)
    return out.reshape(prefix + (OUT_DIM,))


# speedup vs baseline: 1.0234x; 1.0234x over previous
import functools

import jax
import jax.numpy as jnp
import numpy as np
from jax.experimental import pallas as pl
from jax.experimental.pallas import tpu as pltpu
from jax.experimental.pallas import tpu_sc as plsc

HEADS = 4
K_DIM = 256
KNN = 32
N_KEYS = 512
IN_DIM = 1024
OUT_DIM = 512
HALF = K_DIM // 2

BLK = 128
NEG = np.float32(-3.0e38)
BIGI = np.int32(2**30)

_NB = [KNN // (a + 1) for a in range(KNN)]
_NCAND = sum(_NB)
_CPAD = 128 - _NCAND


def _argmax_tree(v, i, aux=None):
    n = v.shape[0]
    while n > 1:
        h = n // 2
        take = v[:h] >= v[h:]
        v = jnp.where(take, v[:h], v[h:])
        i = jnp.where(take, i[:h], i[h:])
        if aux is not None:
            aux = jnp.where(take, aux[:h], aux[h:])
        n = h
    return (v, i) if aux is None else (v, i, aux)


def _topk_iter(s, k, iota):
    vals, idxs = [], []
    for _ in range(k):
        m, pos = _argmax_tree(s, iota)
        vals.append(m)
        idxs.append(pos)
        s = jnp.where(iota == pos, NEG, s)
    return jnp.concatenate(vals, axis=0), jnp.concatenate(idxs, axis=0)


def _stage_a_body(x_ref, wq_ref, bq_ref, keys_ref, idx_ref, w_ref):
    x = x_ref[...]
    q = jax.lax.dot_general(
        x, wq_ref[...], (((1,), (1,)), ((), ())),
        preferred_element_type=jnp.float32,
    ) + bq_ref[...]

    iota_n = jax.lax.broadcasted_iota(jnp.int32, (N_KEYS, BLK), 0)
    iota_c = jax.lax.broadcasted_iota(jnp.int32, (128, BLK), 0)

    w_heads, i_heads = [], []
    for h in range(HEADS):
        tv, ti = [], []
        for p in range(2):
            off = h * K_DIM + p * HALF
            qh = q[:, off:off + HALF]
            sT = jax.lax.dot_general(
                keys_ref[h, p], qh, (((1,), (1,)), ((), ())),
                preferred_element_type=jnp.float32,
                    )
            v, i = _topk_iter(sT, KNN, iota_n)
            tv.append(v)
            ti.append(i)
        (v1, v2), (i1, i2) = tv, ti
        cv, ci = [], []
        for a in range(KNN):
            nb = _NB[a]
            cv.append(v1[a:a + 1] + v2[:nb])
            ci.append(i1[a:a + 1] * N_KEYS + i2[:nb])
        cv.append(jnp.full((_CPAD, BLK), NEG, jnp.float32))
        ci.append(jnp.zeros((_CPAD, BLK), jnp.int32))
        cv = jnp.concatenate(cv, axis=0)
        ci = jnp.concatenate(ci, axis=0)

        svals, gidxs = [], []
        for _ in range(KNN):
            m, pos, gi = _argmax_tree(cv, iota_c, ci)
            svals.append(m)
            gidxs.append(gi)
            cv = jnp.where(iota_c == pos, NEG, cv)
        sc = jnp.concatenate(svals, axis=0)
        gi = jnp.concatenate(gidxs, axis=0)
        e = jnp.exp(sc - sc[0:1])
        wgt = e / jnp.sum(e, axis=0, keepdims=True)
        w_heads.append(wgt)
        i_heads.append(gi)

    w_all = jnp.concatenate(w_heads, axis=0)
    i_all = jnp.concatenate(i_heads, axis=0)
    idx_ref[...] = i_all.T
    w_ref[...] = w_all.T


def _stage_a(x2d, wq, bq2d, keys):
    bs = x2d.shape[0]
    grid = bs // BLK
    return pl.pallas_call(
        _stage_a_body,
        grid=(grid,),
        in_specs=[
            pl.BlockSpec((BLK, IN_DIM), lambda i: (i, 0)),
            pl.BlockSpec((HEADS * K_DIM, IN_DIM), lambda i: (0, 0)),
            pl.BlockSpec((1, HEADS * K_DIM), lambda i: (0, 0)),
            pl.BlockSpec((HEADS, 2, N_KEYS, HALF), lambda i: (0, 0, 0, 0)),
        ],
        out_specs=[
            pl.BlockSpec((BLK, HEADS * KNN), lambda i: (i, 0)),
            pl.BlockSpec((BLK, HEADS * KNN), lambda i: (i, 0)),
        ],
        out_shape=[
            jax.ShapeDtypeStruct((bs, HEADS * KNN), jnp.int32),
            jax.ShapeDtypeStruct((bs, HEADS * KNN), jnp.float32),
        ],
    )(x2d, wq, bq2d, keys)


NW = 32
F = HEADS * KNN


G = 32
HF = F // 2


def _stage_b(values, idx, w):
    bs = idx.shape[0]
    rpw = bs // NW
    ngrp = rpw // G
    mesh = plsc.VectorSubcoreMesh(core_axis_name="c", subcore_axis_name="s")

    @functools.partial(
        pl.kernel,
        out_type=jax.ShapeDtypeStruct((bs, OUT_DIM), jnp.float32),
        mesh=mesh,
        scratch_types=[
            pltpu.VMEM((G, F), jnp.int32),
            pltpu.VMEM((G, F), jnp.float32),
            pltpu.VMEM((G, OUT_DIM), jnp.float32),
            pltpu.VMEM((HF, OUT_DIM), jnp.float32),
            pltpu.VMEM((HF, OUT_DIM), jnp.float32),
            pltpu.SemaphoreType.DMA,
            pltpu.SemaphoreType.DMA,
        ],
        compiler_params=pltpu.CompilerParams(needs_layout_passes=False),
    )
    def bag(values_hbm, idx_hbm, w_hbm, out_hbm, idx_blk, w_blk, out_blk,
            buf0, buf1, sem0, sem1):
        wid = jax.lax.axis_index("s") * 2 + jax.lax.axis_index("c")
        base = wid * rpw

        def gather(r, h, buf, sem):
            return pltpu.make_async_copy(
                values_hbm.at[idx_blk.at[r, pl.ds(h * HF, HF)]], buf, sem)

        def mac_half(r, joff, buf, acc):
            def mac(j, acc):
                wj = plsc.load_gather(
                    w_blk, [jnp.full((16,), r, jnp.int32),
                            jnp.full((16,), joff + j, jnp.int32)])
                return tuple(
                    acc[c] + wj * buf[j, pl.ds(c * 16, 16)]
                    for c in range(OUT_DIM // 16)
                )
            return jax.lax.fori_loop(0, HF, mac, acc)

        def group_body(g, _):
            gbase = base + g * G
            pltpu.sync_copy(idx_hbm.at[pl.ds(gbase, G)], idx_blk)
            pltpu.sync_copy(w_hbm.at[pl.ds(gbase, G)], w_blk)
            gather(0, 0, buf0, sem0).start()
            gather(0, 1, buf1, sem1).start()

            def row_body(r, _):
                zero = tuple(jnp.zeros((16,), jnp.float32)
                             for _ in range(OUT_DIM // 16))
                gather(r, 0, buf0, sem0).wait()
                acc = mac_half(r, 0, buf0, zero)

                @pl.when(r + 1 < G)
                def _():
                    gather(r + 1, 0, buf0, sem0).start()

                gather(r, 1, buf1, sem1).wait()
                acc = mac_half(r, HF, buf1, acc)

                @pl.when(r + 1 < G)
                def _():
                    gather(r + 1, 1, buf1, sem1).start()

                for c in range(OUT_DIM // 16):
                    out_blk[r, pl.ds(c * 16, 16)] = acc[c]
                return ()

            jax.lax.fori_loop(0, G, row_body, ())
            pltpu.sync_copy(out_blk, out_hbm.at[pl.ds(gbase, G)])
            return ()

        jax.lax.fori_loop(0, ngrp, group_body, ())

    return bag(values, idx, w)


NCHUNK = 8


def kernel(x, Wq, bq, keys, values):
    prefix = x.shape[:-1]
    x2d = x.reshape(-1, IN_DIM)
    bs = x2d.shape[0]
    cs = bs // NCHUNK
    bq2d = bq.reshape(1, -1)
    outs = []
    for c in range(NCHUNK):
        xc = jax.lax.slice_in_dim(x2d, c * cs, (c + 1) * cs, axis=0)
        idx, w = _stage_a(xc, Wq, bq2d, keys)
        outs.append(_stage_b(values, idx, w))
    out = jnp.concatenate(outs, axis=0)
    return out.reshape(prefix + (OUT_DIM,))
